# Initial kernel scaffold; baseline (speedup 1.0000x reference)
import jax
import jax.numpy as jnp
from jax.experimental import pallas as pl


def kernel(region_embeddings, tag_embeddings, tags):
    raise NotImplementedError("write your pallas kernel here")


# speedup vs baseline: 7.3907x; 7.3907x over previous
import functools

import jax
import jax.numpy as jnp
from jax import lax
from jax.experimental import pallas as pl
from jax.experimental.pallas import tpu as pltpu

_N = 1024
_D = 64
_T = 16
_NORM_TEMP = 50.0
_SCALE = 0.1 / 32.0
_KB = 2048


def _dense_body(re_ref, te_ref, partials_ref, out_ref, ren_ref, *, n_blocks, k_total):
    pid = pl.program_id(0)

    @pl.when(pid == 0)
    def _init():
        re = re_ref[...]
        ss = jnp.sum(re * re, axis=1, keepdims=True)
        inv = _NORM_TEMP * lax.rsqrt(jnp.maximum(ss, 1e-24))
        ren_ref[...] = (re * inv).astype(jnp.bfloat16)
        out_ref[0, 0] = 0.0

    te = te_ref[...]
    ss_t = jnp.sum(te * te, axis=1, keepdims=True)
    te_n = (te * lax.rsqrt(jnp.maximum(ss_t, 1e-24))).astype(jnp.bfloat16)
    s = lax.dot_general(ren_ref[...], te_n, (((1,), (1,)), ((), ())),
                        preferred_element_type=jnp.float32)
    sp = jnp.maximum(s, 0.0) + jnp.log1p(jnp.exp(-jnp.abs(s)))
    col = pid * _KB + lax.broadcasted_iota(jnp.int32, (1, _KB), 1)
    sp = jnp.where(col < k_total, sp, 0.0)
    out_ref[0, 0] += jnp.sum(sp)

    @pl.when(pid == n_blocks - 1)
    def _finish():
        label = _NORM_TEMP * jnp.sum(partials_ref[...])
        out_ref[0, 0] = (out_ref[0, 0] - label) * _SCALE


def _label_partials(region_embeddings, tag_embeddings, tags):
    g = tag_embeddings[tags]
    dots = jnp.einsum("ntd,nd->nt", g, region_embeddings)
    ss_te = jnp.sum(g * g, axis=-1)
    ss_re = jnp.sum(region_embeddings * region_embeddings, axis=-1, keepdims=True)
    inv = lax.rsqrt(jnp.maximum(ss_te * ss_re, 1e-30))
    t = jnp.arange(_T)
    eq = tags[:, :, None] == tags[:, None, :]
    dup = jnp.any(eq & (t[None, None, :] < t[None, :, None]), axis=-1)
    contrib = jnp.where(dup, 0.0, dots * inv)
    return contrib.reshape(32, _T, 32).sum(axis=-1)


def kernel(region_embeddings, tag_embeddings, tags):
    k_total = tag_embeddings.shape[0]
    n_blocks = (k_total + _KB - 1) // _KB
    k_pad = n_blocks * _KB
    te = jnp.pad(tag_embeddings, ((0, k_pad - k_total), (0, 0)))

    partials = _label_partials(region_embeddings, tag_embeddings, tags)

    out = pl.pallas_call(
        functools.partial(_dense_body, n_blocks=n_blocks, k_total=k_total),
        grid=(n_blocks,),
        in_specs=[
            pl.BlockSpec((_N, _D), lambda i: (0, 0)),
            pl.BlockSpec((_KB, _D), lambda i: (i, 0)),
            pl.BlockSpec((32, _T), lambda i: (0, 0)),
        ],
        out_specs=pl.BlockSpec(memory_space=pltpu.SMEM),
        out_shape=jax.ShapeDtypeStruct((1, 1), jnp.float32),
        scratch_shapes=[pltpu.VMEM((_N, _D), jnp.bfloat16)],
        compiler_params=pltpu.CompilerParams(
            dimension_semantics=("arbitrary",),
        ),
    )(region_embeddings, te, partials)
    return out[0, 0]


# speedup vs baseline: 9.9871x; 1.3513x over previous
import functools

import jax
import jax.numpy as jnp
from jax import lax
from jax.experimental import pallas as pl
from jax.experimental.pallas import tpu as pltpu

_N = 1024
_D = 64
_T = 16
_NORM_TEMP = 50.0
_SCALE = 0.1 / 32.0
_KB = 4000
_LOG2E = 1.4426950408889634
_LN2 = 0.6931471805599453


def _dense_body(re_ref, te_ref, partials_ref, out_ref, ren_ref, *, n_blocks):
    pid = pl.program_id(0)

    @pl.when(pid == 0)
    def _init():
        re = re_ref[...]
        ss = jnp.sum(re * re, axis=1, keepdims=True)
        inv = (_NORM_TEMP * _LOG2E) * lax.rsqrt(jnp.maximum(ss, 1e-24))
        ren_ref[...] = (re * inv).astype(jnp.bfloat16)
        out_ref[0, 0] = 0.0

    te = te_ref[...]
    ss_t = jnp.sum(te * te, axis=1, keepdims=True)
    te_n = (te * lax.rsqrt(jnp.maximum(ss_t, 1e-24))).astype(jnp.bfloat16)
    s2 = lax.dot_general(ren_ref[...], te_n, (((1,), (1,)), ((), ())),
                         preferred_element_type=jnp.float32)
    out_ref[0, 0] += jnp.sum(jnp.log2(1.0 + jnp.exp2(s2)))

    @pl.when(pid == n_blocks - 1)
    def _finish():
        label = _NORM_TEMP * jnp.sum(partials_ref[...])
        out_ref[0, 0] = (out_ref[0, 0] * _LN2 - label) * _SCALE


def _label_partials(region_embeddings, tag_embeddings, tags):
    g = tag_embeddings[tags]
    dots = jnp.einsum("ntd,nd->nt", g, region_embeddings)
    ss_te = jnp.sum(g * g, axis=-1)
    ss_re = jnp.sum(region_embeddings * region_embeddings, axis=-1, keepdims=True)
    inv = lax.rsqrt(jnp.maximum(ss_te * ss_re, 1e-30))
    t = jnp.arange(_T)
    eq = tags[:, :, None] == tags[:, None, :]
    dup = jnp.any(eq & (t[None, None, :] < t[None, :, None]), axis=-1)
    contrib = jnp.where(dup, 0.0, dots * inv)
    return contrib.reshape(32, _T, 32).sum(axis=-1)


def kernel(region_embeddings, tag_embeddings, tags):
    k_total = tag_embeddings.shape[0]
    n_blocks = k_total // _KB
    assert n_blocks * _KB == k_total

    partials = _label_partials(region_embeddings, tag_embeddings, tags)

    out = pl.pallas_call(
        functools.partial(_dense_body, n_blocks=n_blocks),
        grid=(n_blocks,),
        in_specs=[
            pl.BlockSpec((_N, _D), lambda i: (0, 0)),
            pl.BlockSpec((_KB, _D), lambda i: (i, 0)),
            pl.BlockSpec((32, _T), lambda i: (0, 0)),
        ],
        out_specs=pl.BlockSpec(memory_space=pltpu.SMEM),
        out_shape=jax.ShapeDtypeStruct((1, 1), jnp.float32),
        scratch_shapes=[pltpu.VMEM((_N, _D), jnp.bfloat16)],
        compiler_params=pltpu.CompilerParams(
            dimension_semantics=("arbitrary",),
        ),
    )(region_embeddings, tag_embeddings, partials)
    return out[0, 0]
